# Initial kernel scaffold; baseline (speedup 1.0000x reference)
import jax
import jax.numpy as jnp
from jax.experimental import pallas as pl


def kernel(x, edge_index, W1, b1, W2, b2, W_out, b_out, Wd, bd, alpha, gamma):
    raise NotImplementedError("write your pallas kernel here")


# speedup vs baseline: 4.4258x; 4.4258x over previous
import functools

import jax
import jax.numpy as jnp
from jax import lax
from jax.experimental import pallas as pl
from jax.experimental.pallas import tpu as pltpu
from jax.experimental.pallas import tpu_sc as plsc

_N = 10000
_E = 320000
_D_IN = 128
_HID = 64
_OUT = 16

_NC = 2
_NS = 16
_NW = _NC * _NS
_EPW = _E // _NW
_C = 80
_NCHUNK = _EPW // _C

_DEGF = 8


def _make_spmm(F, gather=True):
  mesh = plsc.VectorSubcoreMesh(core_axis_name="c", subcore_axis_name="s")
  scratch = [
      pltpu.VMEM((_C,), jnp.int32),
      pltpu.VMEM((_C,), jnp.int32),
      pltpu.VMEM((_C, F), jnp.float32),
      pltpu.VMEM_SHARED((_N, F), jnp.float32),
      pltpu.SemaphoreType.DMA,
  ]

  @functools.partial(
      pl.kernel,
      out_type=jax.ShapeDtypeStruct((_NC, _N, F), jnp.float32),
      mesh=mesh,
      scratch_types=scratch,
      compiler_params=pltpu.CompilerParams(use_tc_tiling_on_sc=False),
  )
  def spmm(table_hbm, src_hbm, dst_hbm, zeros_hbm, out_hbm,
           src_v, dst_v, rows_v, acc_sh, sem):
    c = lax.axis_index("c")
    s = lax.axis_index("s")
    wid = c * _NS + s

    @pl.when(s == 0)
    def _zero():
      pltpu.sync_copy(zeros_hbm, acc_sh)

    if not gather:
      pltpu.sync_copy(table_hbm, rows_v)
    plsc.subcore_barrier()

    base = wid * _EPW

    def chunk(j, carry):
      off = base + j * _C
      pltpu.sync_copy(dst_hbm.at[pl.ds(off, _C)], dst_v)
      if gather:
        pltpu.sync_copy(src_hbm.at[pl.ds(off, _C)], src_v)
        pltpu.async_copy(table_hbm.at[src_v], rows_v, sem).wait()
      pltpu.sync_copy(rows_v, acc_sh.at[dst_v], add=True)
      return carry

    lax.fori_loop(0, _NCHUNK, chunk, 0)
    plsc.subcore_barrier()

    @pl.when(s == 0)
    def _dump():
      pltpu.sync_copy(acc_sh, out_hbm.at[c])

  return spmm


_spmm128 = _make_spmm(_D_IN, gather=True)
_spmm64 = _make_spmm(_HID, gather=True)
_deg_hist = _make_spmm(_DEGF, gather=False)


def _norm(z):
  n = jnp.sqrt(jnp.sum(z * z, axis=1, keepdims=True))
  return z / jnp.maximum(n, 1e-12)


def _leaky(z):
  return jnp.where(z >= 0, z, 0.01 * z)


def _softmax_head(alpha_row, gamma_s):
  m = jnp.max(alpha_row)
  e = jnp.exp(alpha_row - m)
  return gamma_s * e / jnp.sum(e)


def _tc1_body(x_ref, w1_ref, b1_ref, degp_ref, y_ref, ds_ref):
  deg = degp_ref[0, :, 0:1] + degp_ref[1, :, 0:1]
  ds = jax.lax.rsqrt(jnp.maximum(deg, 1.0))
  h = jnp.dot(x_ref[...], w1_ref[...],
              preferred_element_type=jnp.float32) + b1_ref[...]
  y_ref[...] = ds * h
  ds_ref[...] = ds


def _tc2_body(pa_ref, ds_ref, w_ref, b_ref, al_ref, g_ref,
              z0_ref, t_ref, acc_ref):
  a = _softmax_head(al_ref[0, :], g_ref[0, 0])
  sa = pa_ref[0] + pa_ref[1]
  ds = ds_ref[...]
  z0 = _leaky(_norm(ds * sa[:, :_HID]))
  z0_ref[...] = z0
  t_ref[...] = sa[:, _HID:]
  head = jnp.dot(z0, w_ref[...], preferred_element_type=jnp.float32) + b_ref[...]
  acc_ref[...] = a[0] * _norm(head)


def _tc3_body(pb_ref, ds_ref, z0_ref, w_ref, b_ref, w2_ref, b2_ref,
              al_ref, g_ref, acc_in_ref, yc_ref, dp_ref, acc_ref):
  a = _softmax_head(al_ref[0, :], g_ref[0, 0])
  sb = pb_ref[0] + pb_ref[1]
  ds = ds_ref[...]
  z1 = _leaky(_norm(0.5 * ds * sb))
  head = jnp.dot(z1, w_ref[...], preferred_element_type=jnp.float32) + b_ref[...]
  x_cat = jnp.concatenate([z0_ref[...], z1], axis=1)
  dp = jnp.dot(x_cat, w2_ref[...], preferred_element_type=jnp.float32) + b2_ref[...]
  dp_ref[...] = dp
  yc_ref[...] = ds * dp
  acc_ref[...] = acc_in_ref[...] + a[1] * _norm(head)


def _tc4_body(pc_ref, ds_ref, dp_ref, w_ref, b_ref, wd_ref, bd_ref,
              al_ref, g_ref, acc_in_ref, y_ref):
  a = _softmax_head(al_ref[0, :], g_ref[0, 0])
  sc = pc_ref[0] + pc_ref[1]
  ds = ds_ref[...]
  z = ds * sc + dp_ref[...]
  z2 = _leaky(_norm(z))
  head = jnp.dot(z2, w_ref[...], preferred_element_type=jnp.float32) + b_ref[...]
  out = acc_in_ref[...] + a[2] * _norm(head)
  out = out + a[3] * (ds * wd_ref[...] + bd_ref[...])
  m = jnp.max(out, axis=1, keepdims=True)
  sh = out - m
  y_ref[...] = sh - jnp.log(jnp.sum(jnp.exp(sh), axis=1, keepdims=True))


def _tc(body, out_shapes):
  return pl.pallas_call(body, out_shape=out_shapes)


def kernel(x, edge_index, W1, b1, W2, b2, W_out, b_out, Wd, bd, alpha, gamma):
  f32 = jnp.float32
  src = edge_index[0]
  dst = edge_index[1]
  zeros_deg = jnp.zeros((_N, _DEGF), f32)
  zeros64 = jnp.zeros((_N, _HID), f32)
  zeros128 = jnp.zeros((_N, _D_IN), f32)
  ones_rows = jnp.ones((_C, _DEGF), f32)
  al_row = alpha.reshape(1, 4).astype(f32)
  g_sc = jnp.reshape(gamma, (1, 1)).astype(f32)

  degp = _deg_hist(ones_rows, src, dst, zeros_deg)

  y128, ds = _tc(_tc1_body, (
      jax.ShapeDtypeStruct((_N, _D_IN), f32),
      jax.ShapeDtypeStruct((_N, 1), f32),
  ))(x, W1, b1.reshape(1, _D_IN), degp)

  pa = _spmm128(y128, src, dst, zeros128)

  z0, t, acc1 = _tc(_tc2_body, (
      jax.ShapeDtypeStruct((_N, _HID), f32),
      jax.ShapeDtypeStruct((_N, _HID), f32),
      jax.ShapeDtypeStruct((_N, _OUT), f32),
  ))(pa, ds, W_out[0], b_out[0].reshape(1, _OUT), al_row, g_sc)

  pb = _spmm64(t, src, dst, zeros64)

  yc, dp, acc2 = _tc(_tc3_body, (
      jax.ShapeDtypeStruct((_N, _HID), f32),
      jax.ShapeDtypeStruct((_N, _HID), f32),
      jax.ShapeDtypeStruct((_N, _OUT), f32),
  ))(pb, ds, z0, W_out[1], b_out[1].reshape(1, _OUT), W2,
     b2.reshape(1, _HID), al_row, g_sc, acc1)

  pc = _spmm64(yc, src, dst, zeros64)

  y_hat = _tc(_tc4_body, jax.ShapeDtypeStruct((_N, _OUT), f32))(
      pc, ds, dp, W_out[0], b_out[0].reshape(1, _OUT), Wd,
      bd.reshape(1, _OUT), al_row, g_sc, acc2)

  return y_hat


# speedup vs baseline: 8.3985x; 1.8976x over previous
import functools

import jax
import jax.numpy as jnp
from jax import lax
from jax.experimental import pallas as pl
from jax.experimental.pallas import tpu as pltpu
from jax.experimental.pallas import tpu_sc as plsc

_N = 10000
_E = 320000
_D_IN = 128
_HID = 64
_OUT = 16

_NC = 2
_NS = 16
_NW = _NC * _NS
_EPW = _E // _NW
_C = 40
_K = 5
_CPW = _EPW // _C
_NBATCH = _CPW // _K
_NPAIR = _NBATCH // 2

_DEGF = 8


def _make_spmm(F, mode):
  gather = mode != "deg"
  dual = mode == "dual"
  cpw = (_E // _C) // (_NS if dual else _NW)
  nbatch = cpw // _K
  npair = nbatch // 2
  mesh = plsc.VectorSubcoreMesh(core_axis_name="c", subcore_axis_name="s")
  scratch = [
      pltpu.VMEM((2, _K, _C), jnp.int32),
      pltpu.VMEM((2, _K, _C), jnp.int32),
      pltpu.VMEM((2, _K, _C, F), jnp.float32),
      pltpu.VMEM_SHARED((_N, F), jnp.float32),
      pltpu.SemaphoreType.DMA,
      pltpu.SemaphoreType.DMA,
      pltpu.SemaphoreType.DMA,
      pltpu.SemaphoreType.DMA,
  ]

  @functools.partial(
      pl.kernel,
      out_type=jax.ShapeDtypeStruct((_NC, _N, F), jnp.float32),
      mesh=mesh,
      scratch_types=scratch,
      compiler_params=pltpu.CompilerParams(use_tc_tiling_on_sc=False),
  )
  def spmm(table_hbm, src_hbm, dst_hbm, zeros_hbm, out_hbm,
           src_b, dst_b, rows_b, acc_sh, g0, g1, s0, s1):
    c = lax.axis_index("c")
    s = lax.axis_index("s")
    gsem = (g0, g1)
    ssem = (s0, s1)
    row0 = (s if dual else c * _NS + s) * cpw

    @pl.when(s == 0)
    def _zero():
      pltpu.sync_copy(zeros_hbm, acc_sh)

    if not gather:
      pltpu.sync_copy(table_hbm, rows_b.at[0, 0])

    def load_idx(st, batch):
      r = row0 + batch * _K
      pltpu.sync_copy(dst_hbm.at[pl.ds(r, _K)], dst_b.at[st])
      if gather:
        if dual:
          pltpu.sync_copy(src_hbm.at[c, pl.ds(r, _K)], src_b.at[st])
        else:
          pltpu.sync_copy(src_hbm.at[pl.ds(r, _K)], src_b.at[st])

    def fire_gathers(st):
      if gather:
        for b in range(_K):
          pltpu.async_copy(table_hbm.at[src_b.at[st, b]],
                           rows_b.at[st, b], gsem[st])

    def drain_gathers(st):
      if gather:
        for b in range(_K):
          pltpu.make_async_copy(table_hbm.at[src_b.at[st, b]],
                                rows_b.at[st, b], gsem[st]).wait()

    def fire_scatters(st):
      for b in range(_K):
        rows = rows_b.at[st, b] if gather else rows_b.at[0, 0]
        pltpu.async_copy(rows, acc_sh.at[dst_b.at[st, b]], ssem[st], add=True)

    def drain_scatters(st):
      for b in range(_K):
        rows = rows_b.at[st, b] if gather else rows_b.at[0, 0]
        pltpu.make_async_copy(rows, acc_sh.at[dst_b.at[st, b]],
                              ssem[st]).wait()

    plsc.subcore_barrier()

    load_idx(0, 0)
    fire_gathers(0)
    load_idx(1, 1)
    fire_gathers(1)

    def body(j, carry):
      drain_gathers(0)
      fire_scatters(0)
      drain_gathers(1)
      fire_scatters(1)

      @pl.when(j < npair - 1)
      def _next():
        drain_scatters(0)
        load_idx(0, 2 * j + 2)
        fire_gathers(0)
        drain_scatters(1)
        load_idx(1, 2 * j + 3)
        fire_gathers(1)

      return carry

    lax.fori_loop(0, npair, body, 0)
    drain_scatters(0)
    drain_scatters(1)

    plsc.subcore_barrier()

    @pl.when(s == 0)
    def _dump():
      pltpu.sync_copy(acc_sh, out_hbm.at[c])

  return spmm


_spmm_dual = _make_spmm(_HID, "dual")
_spmm64 = _make_spmm(_HID, "split")
_deg_hist = _make_spmm(_DEGF, "deg")


def _norm(z):
  n = jnp.sqrt(jnp.sum(z * z, axis=1, keepdims=True))
  return z / jnp.maximum(n, 1e-12)


def _leaky(z):
  return jnp.where(z >= 0, z, 0.01 * z)


def _softmax_head(alpha_row, gamma_s):
  m = jnp.max(alpha_row)
  e = jnp.exp(alpha_row - m)
  return gamma_s * e / jnp.sum(e)


def _tc1_body(x_ref, w1_ref, b1_ref, degp_ref, y_ref, ds_ref):
  deg = degp_ref[0, :, 0:1] + degp_ref[1, :, 0:1]
  ds = jax.lax.rsqrt(jnp.maximum(deg, 1.0))
  h = jnp.dot(x_ref[...], w1_ref[...],
              preferred_element_type=jnp.float32) + b1_ref[...]
  y = ds * h
  y_ref[...] = jnp.concatenate([y[:, :_HID], y[:, _HID:]], axis=0)
  ds_ref[...] = ds


def _tc2_body(pa_ref, ds_ref, w_ref, b_ref, al_ref, g_ref,
              z0_ref, acc_ref):
  a = _softmax_head(al_ref[0, :], g_ref[0, 0])
  ds = ds_ref[...]
  z0 = _leaky(_norm(ds * pa_ref[0]))
  z0_ref[...] = z0
  head = jnp.dot(z0, w_ref[...], preferred_element_type=jnp.float32) + b_ref[...]
  acc_ref[...] = a[0] * _norm(head)


def _tc3_body(pb_ref, ds_ref, z0_ref, w_ref, b_ref, w2_ref, b2_ref,
              al_ref, g_ref, acc_in_ref, yc_ref, dp_ref, acc_ref):
  a = _softmax_head(al_ref[0, :], g_ref[0, 0])
  sb = pb_ref[0] + pb_ref[1]
  ds = ds_ref[...]
  z1 = _leaky(_norm(0.5 * ds * sb))
  head = jnp.dot(z1, w_ref[...], preferred_element_type=jnp.float32) + b_ref[...]
  x_cat = jnp.concatenate([z0_ref[...], z1], axis=1)
  dp = jnp.dot(x_cat, w2_ref[...], preferred_element_type=jnp.float32) + b2_ref[...]
  dp_ref[...] = dp
  yc_ref[...] = ds * dp
  acc_ref[...] = acc_in_ref[...] + a[1] * _norm(head)


def _tc4_body(pc_ref, ds_ref, dp_ref, w_ref, b_ref, wd_ref, bd_ref,
              al_ref, g_ref, acc_in_ref, y_ref):
  a = _softmax_head(al_ref[0, :], g_ref[0, 0])
  sc = pc_ref[0] + pc_ref[1]
  ds = ds_ref[...]
  z = ds * sc + dp_ref[...]
  z2 = _leaky(_norm(z))
  head = jnp.dot(z2, w_ref[...], preferred_element_type=jnp.float32) + b_ref[...]
  out = acc_in_ref[...] + a[2] * _norm(head)
  out = out + a[3] * (ds * wd_ref[...] + bd_ref[...])
  m = jnp.max(out, axis=1, keepdims=True)
  sh = out - m
  y_ref[...] = sh - jnp.log(jnp.sum(jnp.exp(sh), axis=1, keepdims=True))


def _tc(body, out_shapes):
  return pl.pallas_call(body, out_shape=out_shapes)


def kernel(x, edge_index, W1, b1, W2, b2, W_out, b_out, Wd, bd, alpha, gamma):
  f32 = jnp.float32
  src = edge_index[0]
  dst = edge_index[1]
  zeros_deg = jnp.zeros((_N, _DEGF), f32)
  zeros64 = jnp.zeros((_N, _HID), f32)
  ones_rows = jnp.ones((_C, _DEGF), f32)
  src = src.reshape(_E // _C, _C)
  dst = dst.reshape(_E // _C, _C)
  al_row = alpha.reshape(1, 4).astype(f32)
  g_sc = jnp.reshape(gamma, (1, 1)).astype(f32)

  src_dual = jnp.stack([src, src + _N])

  degp = _deg_hist(ones_rows, src, dst, zeros_deg)

  y2n, ds = _tc(_tc1_body, (
      jax.ShapeDtypeStruct((2 * _N, _HID), f32),
      jax.ShapeDtypeStruct((_N, 1), f32),
  ))(x, W1, b1.reshape(1, _D_IN), degp)

  pa = _spmm_dual(y2n, src_dual, dst, zeros64)

  z0, acc1 = _tc(_tc2_body, (
      jax.ShapeDtypeStruct((_N, _HID), f32),
      jax.ShapeDtypeStruct((_N, _OUT), f32),
  ))(pa, ds, W_out[0], b_out[0].reshape(1, _OUT), al_row, g_sc)

  pb = _spmm64(pa[1], src, dst, zeros64)

  yc, dp, acc2 = _tc(_tc3_body, (
      jax.ShapeDtypeStruct((_N, _HID), f32),
      jax.ShapeDtypeStruct((_N, _HID), f32),
      jax.ShapeDtypeStruct((_N, _OUT), f32),
  ))(pb, ds, z0, W_out[1], b_out[1].reshape(1, _OUT), W2,
     b2.reshape(1, _HID), al_row, g_sc, acc1)

  pc = _spmm64(yc, src, dst, zeros64)

  y_hat = _tc(_tc4_body, jax.ShapeDtypeStruct((_N, _OUT), f32))(
      pc, ds, dp, W_out[0], b_out[0].reshape(1, _OUT), Wd,
      bd.reshape(1, _OUT), al_row, g_sc, acc2)

  return y_hat
